# baseline (speedup 1.0000x reference)
import functools

import jax
import jax.numpy as jnp
from jax import lax
from jax.experimental import pallas as pl
from jax.experimental.pallas import tpu as pltpu
from jax.experimental.pallas import tpu_sc as plsc

B = 32
S = 8192
NQ = 16384
DX = 64
DZ = 32
NV = S // 128
NVQ = NQ // 128
OCROWS = 128
NOC = NQ * 8 // (OCROWS * 128)
GPO = OCROWS * 128 // 8 // 16

_MESH = plsc.VectorSubcoreMesh(core_axis_name="c", subcore_axis_name="s")


def _gather_body(xq, zq, idxq, xgq, zgq, idxbuf, chunk, outbuf, wsem0, wsem1):
    nc = 2
    b = lax.axis_index("s") * nc + lax.axis_index("c")
    tb = b // 8
    rb = b % 8
    wsems = (wsem0, wsem1)

    pltpu.sync_copy(idxq.at[tb, :, pl.ds(rb, 1), :], idxbuf)

    @pl.loop(0, NVQ)
    def _(vq):
        for g in range(8):
            s = idxbuf[vq, 0, pl.ds(g * 16, 16)]
            idxbuf[vq, 0, pl.ds(g * 16, 16)] = ((s >> 7) << 10) | (s & 127)

    def run_job(src, dst, job, first):
        pltpu.sync_copy(src.at[pl.ds(job * 512, 512)], chunk)
        obase = job * (NOC * OCROWS)

        @pl.loop(0, NOC, step=2)
        def _(oc2):
            for p in range(2):
                oc = oc2 + p
                cond = (oc >= 2) if first else (oc >= 0)

                @pl.when(cond)
                def _():
                    pltpu.make_async_copy(
                        outbuf.at[p],
                        dst.at[pl.ds(obase, OCROWS)],
                        wsems[p]).wait()

                @plsc.parallel_loop(0, GPO, unroll=4)
                def _(ig):
                    grp = oc * GPO + ig
                    vq = grp >> 3
                    g = grp & 7
                    pk = idxbuf[vq, 0, pl.ds(g * 16, 16)]
                    rv = pk >> 7
                    lv = pk & 127
                    lrow = (ig >> 3) * 8
                    lg = ig & 7
                    for r in range(8):
                        vals = plsc.load_gather(chunk, [rv + r, lv])
                        outbuf[p, lrow + r, pl.ds(lg * 16, 16)] = vals

                pltpu.async_copy(
                    outbuf.at[p],
                    dst.at[pl.ds(obase + oc * OCROWS, OCROWS)],
                    wsems[p])

    for u in range(8):
        run_job(xq, xgq, b * 8 + u, first=(u == 0))
    for w in range(4):
        run_job(zq, zgq, b * 4 + w, first=False)

    for p in range(2):
        pltpu.make_async_copy(
            outbuf.at[p],
            zgq.at[pl.ds(0, OCROWS)],
            wsems[p]).wait()


@jax.jit
def _run(xq, zq, idxq):
    return pl.kernel(
        _gather_body,
        out_type=(
            jax.ShapeDtypeStruct((B * 8 * NVQ * 8, 128), jnp.float32),
            jax.ShapeDtypeStruct((B * 4 * NVQ * 8, 128), jnp.float32),
        ),
        mesh=_MESH,
        scratch_types=[
            pltpu.VMEM((NVQ, 1, 128), jnp.int32),
            pltpu.VMEM((512, 128), jnp.float32),
            pltpu.VMEM((2, OCROWS, 128), jnp.float32),
            pltpu.SemaphoreType.DMA,
            pltpu.SemaphoreType.DMA,
        ],
        compiler_params=pltpu.CompilerParams(
            use_tc_tiling_on_sc=False, needs_layout_passes=False),
    )(xq, zq, idxq)


def kernel(x, z, idx_pa):
    xq = (x.transpose(0, 2, 1)
           .reshape(B, 8, 8, NV, 128)
           .transpose(0, 1, 3, 2, 4)
           .reshape(B * 8 * NV * 8, 128))
    zq = (z.transpose(0, 2, 1)
           .reshape(B, 4, 8, NV, 128)
           .transpose(0, 1, 3, 2, 4)
           .reshape(B * 4 * NV * 8, 128))
    idxq = (idx_pa.astype(jnp.int32)
            .reshape(4, 8, NVQ, 128)
            .transpose(0, 2, 1, 3))
    xgq, zgq = _run(xq, zq, idxq)
    xg = (xgq.reshape(B, 8, NVQ, 8, 128)
             .transpose(0, 1, 3, 2, 4)
             .reshape(B, DX, NQ)
             .transpose(0, 2, 1))
    zg = (zgq.reshape(B, 4, NVQ, 8, 128)
             .transpose(0, 1, 3, 2, 4)
             .reshape(B, DZ, NQ)
             .transpose(0, 2, 1))
    return xg, zg


# speedup vs baseline: 1.2193x; 1.2193x over previous
import functools

import jax
import jax.numpy as jnp
from jax import lax
from jax.experimental import pallas as pl
from jax.experimental.pallas import tpu as pltpu
from jax.experimental.pallas import tpu_sc as plsc

B = 32
S = 8192
NQ = 16384
DX = 64
DZ = 32
NV = S // 128
NVQ = NQ // 128
VJB = 16
NOC = NVQ // VJB
GPO = VJB * 8

_MESH = plsc.VectorSubcoreMesh(core_axis_name="c", subcore_axis_name="s")


def _gather_body(xq, zq, idxq, xgq, zgq,
                 idxbuf, chunk0, chunk1, outbuf,
                 csem0, csem1, wsem0, wsem1):
    nc = 2
    b = lax.axis_index("s") * nc + lax.axis_index("c")
    tb = b // 8
    rb = b % 8
    chunks = (chunk0, chunk1)
    csems = (csem0, csem1)
    wsems = (wsem0, wsem1)

    pltpu.sync_copy(idxq.at[tb, :, pl.ds(rb, 1), :], idxbuf)

    @pl.loop(0, NVQ)
    def _(vq):
        for g in range(8):
            s = idxbuf[vq, 0, pl.ds(g * 16, 16)]
            idxbuf[vq, 0, pl.ds(g * 16, 16)] = ((s >> 7) << 9) | (s & 127)

    rfull = [jax.lax.broadcast(jnp.int32(r), (16,)) for r in range(4)]

    hjobs = []
    for u in range(8):
        for h in range(2):
            hjobs.append((xq, xgq, b * 8 + u, h))
    for w in range(4):
        for h in range(2):
            hjobs.append((zq, zgq, b * 4 + w, h))

    def load(i, cb):
        src, _, job, h = hjobs[i]
        return pltpu.async_copy(
            src.at[job, :, pl.ds(h * 4, 4), :], chunks[cb], csems[cb])

    load(0, 0)
    for i, (src, dst, job, h) in enumerate(hjobs):
        cb = i % 2
        chunk = chunks[cb]
        pltpu.make_async_copy(
            src.at[job, :, pl.ds(h * 4, 4), :], chunk, csems[cb]).wait()
        if i + 1 < len(hjobs):
            load(i + 1, 1 - cb)

        @pl.loop(0, NOC, step=2)
        def _(oc2):
            for p in range(2):
                oc = oc2 + p
                cond = (oc >= 2) if i == 0 else (oc >= 0)

                @pl.when(cond)
                def _():
                    pltpu.make_async_copy(
                        outbuf.at[p],
                        dst.at[0, pl.ds(0, VJB), pl.ds(0, 1), :, :],
                        wsems[p]).wait()

                @plsc.parallel_loop(0, GPO, unroll=4)
                def _(ig):
                    grp = oc * GPO + ig
                    vq = grp >> 3
                    g = grp & 7
                    pk = idxbuf[vq, 0, pl.ds(g * 16, 16)]
                    vv = pk >> 9
                    lv = pk & 127
                    vjl = ig >> 3
                    lg = ig & 7
                    for r in range(4):
                        vals = plsc.load_gather(chunk, [vv, rfull[r], lv])
                        outbuf[p, vjl, 0, r, pl.ds(lg * 16, 16)] = vals

                pltpu.async_copy(
                    outbuf.at[p],
                    dst.at[job, pl.ds(oc * VJB, VJB), pl.ds(h, 1), :, :],
                    wsems[p])

    for p in range(2):
        pltpu.make_async_copy(
            outbuf.at[p],
            zgq.at[0, pl.ds(0, VJB), pl.ds(0, 1), :, :],
            wsems[p]).wait()


@jax.jit
def _run(xq, zq, idxq):
    return pl.kernel(
        _gather_body,
        out_type=(
            jax.ShapeDtypeStruct((B * 8, NVQ, 2, 4, 128), jnp.float32),
            jax.ShapeDtypeStruct((B * 4, NVQ, 2, 4, 128), jnp.float32),
        ),
        mesh=_MESH,
        scratch_types=[
            pltpu.VMEM((NVQ, 1, 128), jnp.int32),
            pltpu.VMEM((NV, 4, 128), jnp.float32),
            pltpu.VMEM((NV, 4, 128), jnp.float32),
            pltpu.VMEM((2, VJB, 1, 4, 128), jnp.float32),
            pltpu.SemaphoreType.DMA,
            pltpu.SemaphoreType.DMA,
            pltpu.SemaphoreType.DMA,
            pltpu.SemaphoreType.DMA,
        ],
        compiler_params=pltpu.CompilerParams(
            use_tc_tiling_on_sc=False, needs_layout_passes=False),
    )(xq, zq, idxq)


def kernel(x, z, idx_pa):
    xq = (x.transpose(0, 2, 1)
           .reshape(B, 8, 8, NV, 128)
           .transpose(0, 1, 3, 2, 4)
           .reshape(B * 8, NV, 8, 128))
    zq = (z.transpose(0, 2, 1)
           .reshape(B, 4, 8, NV, 128)
           .transpose(0, 1, 3, 2, 4)
           .reshape(B * 4, NV, 8, 128))
    idxq = (idx_pa.astype(jnp.int32)
            .reshape(4, 8, NVQ, 128)
            .transpose(0, 2, 1, 3))
    xgq, zgq = _run(xq, zq, idxq)
    xg = (xgq.reshape(B, 8, NVQ, 8, 128)
             .transpose(0, 1, 3, 2, 4)
             .reshape(B, DX, NQ)
             .transpose(0, 2, 1))
    zg = (zgq.reshape(B, 4, NVQ, 8, 128)
             .transpose(0, 1, 3, 2, 4)
             .reshape(B, DZ, NQ)
             .transpose(0, 2, 1))
    return xg, zg
